# baseline (speedup 1.0000x reference)
import functools

import jax
import jax.numpy as jnp
from jax.experimental import pallas as pl


def _reduce_body(x_ref, o_ref, *, scale):
    m = pl.program_id(1)
    s = jnp.sum(x_ref[...], axis=(1, 3, 4), keepdims=False)
    s = s[:, None, :] * scale

    @pl.when(m == 0)
    def _init():
        o_ref[...] = s

    @pl.when(m != 0)
    def _acc():
        o_ref[...] += s


def kernel(x, protos):
    N, M, C, T, V = x.shape
    P = protos.shape[0]
    scale = 1.0 / (M * P)

    out = pl.pallas_call(
        functools.partial(_reduce_body, scale=scale),
        grid=(N, M),
        in_specs=[pl.BlockSpec((1, 1, C, T, V), lambda n, m: (n, m, 0, 0, 0))],
        out_specs=pl.BlockSpec((1, 1, C), lambda n, m: (n, 0, 0)),
        out_shape=jax.ShapeDtypeStruct((N, 1, C), x.dtype),
    )(x)
    return out.reshape(N, C)
